# baseline (speedup 1.0000x reference)
import jax
import jax.numpy as jnp
from jax import lax
from jax.experimental import pallas as pl
from jax.experimental.pallas import tpu as pltpu
from jax.experimental.pallas import tpu_sc as plsc

_NC, _NS = 2, 16
_NW = _NC * _NS
_LANES = 16
_CW = 16 * 1024
_NBUF = 3


def _sc_body(total_words):
    n_chunks_total = total_words // (_NW * _CW)

    def body(x_hbm, p_hbm, o_hbm,
             bx0, bx1, bx2, bp0, bp1, bp2, ldsem, stsem):
        bufx = (bx0, bx1, bx2)
        bufp = (bp0, bp1, bp2)
        wid = lax.axis_index("s") * _NC + lax.axis_index("c")
        base = wid * (total_words // _NW)

        def load_descs(j):
            b = j % _NBUF
            off = base + j * _CW
            return (
                pltpu.make_async_copy(
                    x_hbm.at[pl.ds(off, _CW)], bufx[b], ldsem.at[b]),
                pltpu.make_async_copy(
                    p_hbm.at[pl.ds(off, _CW)], bufp[b], ldsem.at[b]),
            )

        def store_desc(j):
            b = j % _NBUF
            off = base + j * _CW
            return pltpu.make_async_copy(
                bufx[b], o_hbm.at[pl.ds(off, _CW)], stsem.at[b])

        for j in range(min(_NBUF - 1, n_chunks_total)):
            for d in load_descs(j):
                d.start()

        stores_waited = set()
        for j in range(n_chunks_total):
            b = j % _NBUF
            for d in load_descs(j):
                d.wait()

            @plsc.parallel_loop(0, _CW, step=_LANES, unroll=8)
            def _(i):
                plsc.addupdate(bufx[b].at[pl.ds(i, _LANES)],
                               bufp[b][pl.ds(i, _LANES)])

            nxt = j + _NBUF - 1
            if nxt < n_chunks_total:
                if j >= 1:
                    store_desc(j - 1).wait()
                    stores_waited.add(j - 1)
                for d in load_descs(nxt):
                    d.start()
            store_desc(j).start()

        for j in range(n_chunks_total):
            if j not in stores_waited:
                store_desc(j).wait()

    return body


def kernel(x, pos_emb):
    S, D = x.shape
    total = S * D
    mesh = plsc.VectorSubcoreMesh(core_axis_name="c", subcore_axis_name="s")
    run = pl.kernel(
        _sc_body(total),
        out_type=jax.ShapeDtypeStruct((total,), jnp.float32),
        mesh=mesh,
        scratch_types=(
            [pltpu.VMEM((_CW,), jnp.float32) for _ in range(2 * _NBUF)]
            + [pltpu.SemaphoreType.DMA((_NBUF,)),
               pltpu.SemaphoreType.DMA((_NBUF,))]
        ),
    )
    out = run(x.reshape(total), pos_emb.reshape(total))
    return out.reshape(1, S, D)
